# Initial kernel scaffold; baseline (speedup 1.0000x reference)
import jax
import jax.numpy as jnp
from jax.experimental import pallas as pl


def kernel(x, edge_index, W1, b1, W2, b2, Wih0, Whh0, bih0, bhh0, Wih1, Whh1, bih1, bhh1):
    raise NotImplementedError("write your pallas kernel here")


# speedup vs baseline: 54.2344x; 54.2344x over previous
import functools

import jax
import jax.numpy as jnp
from jax import lax
from jax.experimental import pallas as pl
from jax.experimental.pallas import tpu as pltpu
from jax.experimental.pallas import tpu_sc as plsc

_N = 10000
_E = 320000
_SEQ = 15
_IN = 128
_NHID = 2
_H = 64

_NCOL1 = 32
_NCOL2 = 16
_NTAB = 10240
_PAD_NODE = 10016
_CHUNK = 4096
_NCHUNK = 82
_EPAD = _NCHUNK * _CHUNK
_NW = 32
_EPG = _EPAD // _NW
_BN = 1000
_NBLK = _N // _BN

@functools.lru_cache(maxsize=None)
def _mesh():
  return plsc.VectorSubcoreMesh(core_axis_name="c", subcore_axis_name="s",
                                num_cores=2, num_subcores=16)


def _wid():
  return lax.axis_index("s") * 2 + lax.axis_index("c")


def _zero_f32(ref, start, count):
  z = jnp.zeros((16,), jnp.float32)

  @pl.loop(0, count)
  def _(i):
    ref[pl.ds(start + i * 16, 16)] = z


@functools.lru_cache(maxsize=None)
def _make_deg():
  @functools.partial(
      pl.kernel,
      out_type=jax.ShapeDtypeStruct((_NW, _N), jnp.float32),
      mesh=_mesh(),
      compiler_params=pltpu.CompilerParams(needs_layout_passes=False, use_tc_tiling_on_sc=False),
      scratch_types=[
          pltpu.VMEM((_EPG,), jnp.int32),
          pltpu.VMEM((_NTAB,), jnp.float32),
      ],
  )
  def deg_kernel(dst_hbm, degp_hbm, dst_v, acc_v):
    wid = _wid()
    _zero_f32(acc_v, 0, _NTAB // 16)
    pltpu.sync_copy(dst_hbm.at[pl.ds(wid * _EPG, _EPG)], dst_v)
    ones = jnp.full((16,), 1.0, jnp.float32)

    @pl.loop(0, _EPG // 16)
    def _(i):
      d = dst_v[pl.ds(i * 16, 16)]
      plsc.addupdate_scatter(acc_v, [d], ones)

    pltpu.sync_copy(acc_v.at[pl.ds(0, _N)], degp_hbm.at[wid])

  return deg_kernel


def _mm_kernel(xb, w1p, hh):
  acc = jnp.zeros((_BN, _NCOL1), jnp.float32)
  for t in range(_SEQ):
    acc = acc + jnp.dot(xb[t], w1p[t], preferred_element_type=jnp.float32)
  hh[...] = acc


def _run_mm(x, w1p):
  return pl.pallas_call(
      _mm_kernel,
      grid=(_NBLK,),
      in_specs=[
          pl.BlockSpec((_SEQ, _BN, _IN), lambda j: (0, j, 0)),
          pl.BlockSpec((_SEQ, _IN, _NCOL1), lambda j: (0, 0, 0)),
      ],
      out_specs=pl.BlockSpec((_BN, _NCOL1), lambda j: (j, 0)),
      out_shape=jax.ShapeDtypeStruct((_N, _NCOL1), jnp.float32),
  )(x, w1p)


def _tr_kernel(hh, degp, hht, dinv):
  hht[...] = hh[...].T
  deg = jnp.sum(degp[...], axis=0, keepdims=True)
  dinv[...] = jnp.where(deg > 0, lax.rsqrt(deg), 0.0)


def _run_tr(hh, degp):
  return pl.pallas_call(
      _tr_kernel,
      out_shape=[
          jax.ShapeDtypeStruct((_NCOL1, _N), jnp.float32),
          jax.ShapeDtypeStruct((1, _N), jnp.float32),
      ],
  )(hh, degp)


@functools.lru_cache(maxsize=None)
def _make_conv(nsplit):
  chunks_per = _NCHUNK // nsplit

  @functools.partial(
      pl.kernel,
      out_type=jax.ShapeDtypeStruct((_NW, _N), jnp.float32),
      mesh=_mesh(),
      compiler_params=pltpu.CompilerParams(needs_layout_passes=False, use_tc_tiling_on_sc=False),
      scratch_types=[
          pltpu.VMEM((_NTAB,), jnp.float32),
          pltpu.VMEM((_NTAB,), jnp.float32),
          pltpu.VMEM((_NTAB,), jnp.float32),
          pltpu.VMEM((_CHUNK,), jnp.int32),
          pltpu.VMEM((_CHUNK,), jnp.int32),
      ],
  )
  def conv(src_hbm, dst_hbm, dinv_hbm, tab_hbm, out_hbm,
           dinv_v, tab_v, acc_v, src_v, dst_v):
    wid = _wid()
    col = wid // nsplit
    part = wid % nsplit
    _zero_f32(acc_v, 0, _NTAB // 16)
    _zero_f32(dinv_v, _N, (_NTAB - _N) // 16)
    _zero_f32(tab_v, _N, (_NTAB - _N) // 16)
    pltpu.sync_copy(dinv_hbm.at[0], dinv_v.at[pl.ds(0, _N)])
    pltpu.sync_copy(tab_hbm.at[col], tab_v.at[pl.ds(0, _N)])

    @pl.loop(0, chunks_per)
    def _(c):
      base = (part * chunks_per + c) * _CHUNK
      pltpu.sync_copy(src_hbm.at[pl.ds(base, _CHUNK)], src_v)
      pltpu.sync_copy(dst_hbm.at[pl.ds(base, _CHUNK)], dst_v)

      @pl.loop(0, _CHUNK // 16)
      def _(i):
        s = src_v[pl.ds(i * 16, 16)]
        d = dst_v[pl.ds(i * 16, 16)]
        nv = plsc.load_gather(dinv_v, [s]) * plsc.load_gather(dinv_v, [d])
        hv = plsc.load_gather(tab_v, [s])
        plsc.addupdate_scatter(acc_v, [d], nv * hv)

    pltpu.sync_copy(acc_v.at[pl.ds(0, _N)], out_hbm.at[wid])

  return conv


def _mid_kernel(msg1, b1col, s2, ut):
  h = jnp.maximum(msg1[...] + b1col[...], 0.0)
  ut[...] = lax.dot_general(
      s2[...], h, (((1,), (0,)), ((), ())),
      preferred_element_type=jnp.float32)


def _run_mid(msg1, b1col, s2):
  return pl.pallas_call(
      _mid_kernel,
      out_shape=jax.ShapeDtypeStruct((_NCOL2, _N), jnp.float32),
  )(msg1, b1col, s2)


def _gru_cell(gi, h, whht, bhh):
  gh = jnp.dot(h, whht[...], preferred_element_type=jnp.float32) + bhh[...]
  r = jax.nn.sigmoid(gi[:, :_H] + gh[:, :_H])
  z = jax.nn.sigmoid(gi[:, _H:2 * _H] + gh[:, _H:2 * _H])
  n = jnp.tanh(gi[:, 2 * _H:] + r * gh[:, 2 * _H:])
  return (1.0 - z) * n + z * h


def _gru_kernel(msg2, b2, wih0t, whh0t, bih0, bhh0, wih1t, whh1t, bih1, bhh1,
                out1, hn):
  rows = [msg2[2 * t:2 * t + 1, :] + msg2[2 * t + 1:2 * t + 2, :]
          for t in range(_SEQ)]
  seq = jnp.tanh(jnp.concatenate(rows, axis=0) + b2[...])
  gi0 = jnp.dot(seq, wih0t[...], preferred_element_type=jnp.float32) + bih0[...]
  h = jnp.zeros((1, _H), jnp.float32)
  outs0 = []
  for t in range(_SEQ):
    h = _gru_cell(gi0[t:t + 1, :], h, whh0t, bhh0)
    outs0.append(h)
  out0 = jnp.concatenate(outs0, axis=0)
  h0T = h
  gi1 = jnp.dot(out0, wih1t[...], preferred_element_type=jnp.float32) + bih1[...]
  h = jnp.zeros((1, _H), jnp.float32)
  outs1 = []
  for t in range(_SEQ):
    h = _gru_cell(gi1[t:t + 1, :], h, whh1t, bhh1)
    outs1.append(h)
  out1[...] = jnp.concatenate(outs1, axis=0)
  hn[...] = jnp.concatenate([h0T, h], axis=0)


def _run_gru(msg2, b2, wih0t, whh0t, bih0, bhh0, wih1t, whh1t, bih1, bhh1):
  return pl.pallas_call(
      _gru_kernel,
      out_shape=[
          jax.ShapeDtypeStruct((_SEQ, _H), jnp.float32),
          jax.ShapeDtypeStruct((2, _H), jnp.float32),
      ],
  )(msg2, b2, wih0t, whh0t, bih0, bhh0, wih1t, whh1t, bih1, bhh1)


def kernel(x, edge_index, W1, b1, W2, b2,
           Wih0, Whh0, bih0, bhh0, Wih1, Whh1, bih1, bhh1):
  loop = jnp.arange(_N, dtype=jnp.int32)
  npad = _EPAD - _E - _N
  src = jnp.concatenate([edge_index[0], loop,
                         jnp.zeros((npad,), jnp.int32)])
  dst = jnp.concatenate([edge_index[1], loop,
                         jnp.full((npad,), _PAD_NODE, jnp.int32)])

  t_ids = jnp.arange(_SEQ, dtype=jnp.int32)
  h_ids = jnp.arange(_NHID, dtype=jnp.int32)
  c_ids = jnp.arange(_NCOL1, dtype=jnp.int32)
  onehot = (c_ids[None, None, :] ==
            (2 * t_ids[:, None, None] + h_ids[None, :, None])
            ).astype(jnp.float32)
  w1p = jnp.einsum("tkh,thc->tkc", W1, onehot)
  s2 = jnp.zeros((_NCOL2, _NCOL1), jnp.float32)
  s2 = s2.at[:_SEQ].set(jnp.einsum("th,thc->tc", W2[:, :, 0], onehot))
  b1col = jnp.zeros((_NCOL1, 1), jnp.float32)
  b1col = b1col.at[:2 * _SEQ, 0].set(b1.reshape(-1))

  degp = _make_deg()(dst)
  hh = _run_mm(x, w1p)
  hht, dinv = _run_tr(hh, degp)
  msg1 = _make_conv(1)(src, dst, dinv, hht)
  ut = _run_mid(msg1, b1col, s2)
  msg2 = _make_conv(2)(src, dst, dinv, ut)
  out1, hn = _run_gru(
      msg2, b2, Wih0.T, Whh0.T, bih0.reshape(1, -1), bhh0.reshape(1, -1),
      Wih1.T, Whh1.T, bih1.reshape(1, -1), bhh1.reshape(1, -1))
  return out1.reshape(_SEQ, 1, _H), hn.reshape(2, 1, _H)


# speedup vs baseline: 78.7188x; 1.4515x over previous
import functools

import jax
import jax.numpy as jnp
from jax import lax
from jax.experimental import pallas as pl
from jax.experimental.pallas import tpu as pltpu
from jax.experimental.pallas import tpu_sc as plsc

_N = 10000
_E = 320000
_SEQ = 15
_IN = 128
_NHID = 2
_H = 64

_NCOL1 = 32
_NCOL2 = 16
_NTAB = 10240
_PAD_NODE = 10016
_CHUNK = 4096
_NCHUNK = 82
_EPAD = _NCHUNK * _CHUNK
_NW = 32
_EPG = _EPAD // _NW
_BN = 1000
_NBLK = _N // _BN

@functools.lru_cache(maxsize=None)
def _mesh():
  return plsc.VectorSubcoreMesh(core_axis_name="c", subcore_axis_name="s",
                                num_cores=2, num_subcores=16)


def _wid():
  return lax.axis_index("s") * 2 + lax.axis_index("c")


def _zero_f32(ref, start, count):
  z = jnp.zeros((16,), jnp.float32)

  @pl.loop(0, count)
  def _(i):
    ref[pl.ds(start + i * 16, 16)] = z


@functools.lru_cache(maxsize=None)
def _make_deg():
  @functools.partial(
      pl.kernel,
      out_type=jax.ShapeDtypeStruct((_NW, _N), jnp.float32),
      mesh=_mesh(),
      compiler_params=pltpu.CompilerParams(needs_layout_passes=False, use_tc_tiling_on_sc=False),
      scratch_types=[
          pltpu.VMEM((_EPG,), jnp.int32),
          pltpu.VMEM((_NTAB,), jnp.float32),
      ],
  )
  def deg_kernel(dst_hbm, degp_hbm, dst_v, acc_v):
    wid = _wid()
    _zero_f32(acc_v, 0, _NTAB // 16)
    pltpu.sync_copy(dst_hbm.at[pl.ds(wid * _EPG, _EPG)], dst_v)
    ones = jnp.full((16,), 1.0, jnp.float32)

    @pl.loop(0, _EPG // 16)
    def _(i):
      d = dst_v[pl.ds(i * 16, 16)]
      plsc.addupdate_scatter(acc_v, [d], ones)

    pltpu.sync_copy(acc_v.at[pl.ds(0, _N)], degp_hbm.at[wid])

  return deg_kernel


def _mm_kernel(xb, w1p, hh):
  acc = jnp.zeros((_BN, _NCOL1), jnp.float32)
  for t in range(_SEQ):
    acc = acc + jnp.dot(xb[t], w1p[t], preferred_element_type=jnp.float32)
  hh[...] = acc


def _run_mm(x, w1p):
  return pl.pallas_call(
      _mm_kernel,
      grid=(_NBLK,),
      in_specs=[
          pl.BlockSpec((_SEQ, _BN, _IN), lambda j: (0, j, 0)),
          pl.BlockSpec((_SEQ, _IN, _NCOL1), lambda j: (0, 0, 0)),
      ],
      out_specs=pl.BlockSpec((_BN, _NCOL1), lambda j: (j, 0)),
      out_shape=jax.ShapeDtypeStruct((_N, _NCOL1), jnp.float32),
  )(x, w1p)


def _tr_kernel(hh, degp, hht, dinv):
  hht[...] = hh[...].T
  deg = jnp.sum(degp[...], axis=0, keepdims=True)
  dinv[...] = jnp.where(deg > 0, lax.rsqrt(deg), 0.0)


def _run_tr(hh, degp):
  return pl.pallas_call(
      _tr_kernel,
      out_shape=[
          jax.ShapeDtypeStruct((_NCOL1, _N), jnp.float32),
          jax.ShapeDtypeStruct((1, _N), jnp.float32),
      ],
  )(hh, degp)


@functools.lru_cache(maxsize=None)
def _make_norm():
  @functools.partial(
      pl.kernel,
      out_type=jax.ShapeDtypeStruct((_EPAD,), jnp.float32),
      mesh=_mesh(),
      compiler_params=pltpu.CompilerParams(needs_layout_passes=False, use_tc_tiling_on_sc=False),
      scratch_types=[
          pltpu.VMEM((_NTAB,), jnp.float32),
          pltpu.VMEM((_EPG,), jnp.int32),
          pltpu.VMEM((_EPG,), jnp.int32),
          pltpu.VMEM((_EPG,), jnp.float32),
      ],
  )
  def norm_kernel(src_hbm, dst_hbm, dinv_hbm, norm_hbm,
                  dinv_v, src_v, dst_v, nrm_v):
    wid = _wid()
    _zero_f32(dinv_v, _N, (_NTAB - _N) // 16)
    pltpu.sync_copy(dinv_hbm.at[0], dinv_v.at[pl.ds(0, _N)])
    pltpu.sync_copy(src_hbm.at[pl.ds(wid * _EPG, _EPG)], src_v)
    pltpu.sync_copy(dst_hbm.at[pl.ds(wid * _EPG, _EPG)], dst_v)

    @pl.loop(0, _EPG // 16, unroll=8)
    def _(i):
      s = src_v[pl.ds(i * 16, 16)]
      d = dst_v[pl.ds(i * 16, 16)]
      nrm_v[pl.ds(i * 16, 16)] = (
          plsc.load_gather(dinv_v, [s]) * plsc.load_gather(dinv_v, [d]))

    pltpu.sync_copy(nrm_v, norm_hbm.at[pl.ds(wid * _EPG, _EPG)])

  return norm_kernel


@functools.lru_cache(maxsize=None)
def _make_conv(nsplit):
  chunks_per = _NCHUNK // nsplit

  @functools.partial(
      pl.kernel,
      out_type=jax.ShapeDtypeStruct((_NW, _N), jnp.float32),
      mesh=_mesh(),
      compiler_params=pltpu.CompilerParams(needs_layout_passes=False, use_tc_tiling_on_sc=False),
      scratch_types=[
          pltpu.VMEM((_NTAB,), jnp.float32),
          pltpu.VMEM((_NTAB,), jnp.float32),
          pltpu.VMEM((2, _CHUNK), jnp.int32),
          pltpu.VMEM((2, _CHUNK), jnp.int32),
          pltpu.VMEM((2, _CHUNK), jnp.float32),
          pltpu.SemaphoreType.DMA,
          pltpu.SemaphoreType.DMA,
      ],
  )
  def conv(src_hbm, dst_hbm, norm_hbm, tab_hbm, out_hbm,
           tab_v, acc_v, src_v, dst_v, nrm_v, sem0, sem1):
    wid = _wid()
    col = wid // nsplit
    part = wid % nsplit
    sems = [sem0, sem1]

    def issue(slot, c):
      base = (part * chunks_per + c) * _CHUNK
      pltpu.async_copy(src_hbm.at[pl.ds(base, _CHUNK)], src_v.at[slot],
                       sems[slot])
      pltpu.async_copy(dst_hbm.at[pl.ds(base, _CHUNK)], dst_v.at[slot],
                       sems[slot])
      pltpu.async_copy(norm_hbm.at[pl.ds(base, _CHUNK)], nrm_v.at[slot],
                       sems[slot])

    def drain(slot, c):
      base = (part * chunks_per + c) * _CHUNK
      pltpu.make_async_copy(src_hbm.at[pl.ds(base, _CHUNK)], src_v.at[slot],
                            sems[slot]).wait()
      pltpu.make_async_copy(dst_hbm.at[pl.ds(base, _CHUNK)], dst_v.at[slot],
                            sems[slot]).wait()
      pltpu.make_async_copy(norm_hbm.at[pl.ds(base, _CHUNK)], nrm_v.at[slot],
                            sems[slot]).wait()

    _zero_f32(acc_v, 0, _NTAB // 16)
    pltpu.sync_copy(tab_hbm.at[col], tab_v.at[pl.ds(0, _N)])
    issue(0, 0)

    @pl.loop(0, (chunks_per + 1) // 2)
    def _(g):
      for b in range(2):
        cidx = g * 2 + b

        @pl.when(cidx + 1 < chunks_per)
        def _():
          issue(1 - b, cidx + 1)

        @pl.when(cidx < chunks_per)
        def _():
          drain(b, cidx)

          @pl.loop(0, _CHUNK // 16, unroll=8)
          def _(i):
            s = src_v[b, pl.ds(i * 16, 16)]
            d = dst_v[b, pl.ds(i * 16, 16)]
            nv = nrm_v[b, pl.ds(i * 16, 16)]
            hv = plsc.load_gather(tab_v, [s])
            plsc.addupdate_scatter(acc_v, [d], nv * hv)

    pltpu.sync_copy(acc_v.at[pl.ds(0, _N)], out_hbm.at[wid])

  return conv


def _mid_kernel(msg1, b1col, s2, ut):
  h = jnp.maximum(msg1[...] + b1col[...], 0.0)
  ut[...] = lax.dot_general(
      s2[...], h, (((1,), (0,)), ((), ())),
      preferred_element_type=jnp.float32)


def _run_mid(msg1, b1col, s2):
  return pl.pallas_call(
      _mid_kernel,
      out_shape=jax.ShapeDtypeStruct((_NCOL2, _N), jnp.float32),
  )(msg1, b1col, s2)


def _gru_cell(gi, h, whht, bhh):
  gh = jnp.dot(h, whht[...], preferred_element_type=jnp.float32) + bhh[...]
  r = jax.nn.sigmoid(gi[:, :_H] + gh[:, :_H])
  z = jax.nn.sigmoid(gi[:, _H:2 * _H] + gh[:, _H:2 * _H])
  n = jnp.tanh(gi[:, 2 * _H:] + r * gh[:, 2 * _H:])
  return (1.0 - z) * n + z * h


def _gru_kernel(msg2, b2, wih0t, whh0t, bih0, bhh0, wih1t, whh1t, bih1, bhh1,
                out1, hn):
  rows = [msg2[2 * t:2 * t + 1, :] + msg2[2 * t + 1:2 * t + 2, :]
          for t in range(_SEQ)]
  seq = jnp.tanh(jnp.concatenate(rows, axis=0) + b2[...])
  gi0 = jnp.dot(seq, wih0t[...], preferred_element_type=jnp.float32) + bih0[...]
  h = jnp.zeros((1, _H), jnp.float32)
  outs0 = []
  for t in range(_SEQ):
    h = _gru_cell(gi0[t:t + 1, :], h, whh0t, bhh0)
    outs0.append(h)
  out0 = jnp.concatenate(outs0, axis=0)
  h0T = h
  gi1 = jnp.dot(out0, wih1t[...], preferred_element_type=jnp.float32) + bih1[...]
  h = jnp.zeros((1, _H), jnp.float32)
  outs1 = []
  for t in range(_SEQ):
    h = _gru_cell(gi1[t:t + 1, :], h, whh1t, bhh1)
    outs1.append(h)
  out1[...] = jnp.concatenate(outs1, axis=0)
  hn[...] = jnp.concatenate([h0T, h], axis=0)


def _run_gru(msg2, b2, wih0t, whh0t, bih0, bhh0, wih1t, whh1t, bih1, bhh1):
  return pl.pallas_call(
      _gru_kernel,
      out_shape=[
          jax.ShapeDtypeStruct((_SEQ, _H), jnp.float32),
          jax.ShapeDtypeStruct((2, _H), jnp.float32),
      ],
  )(msg2, b2, wih0t, whh0t, bih0, bhh0, wih1t, whh1t, bih1, bhh1)


def kernel(x, edge_index, W1, b1, W2, b2,
           Wih0, Whh0, bih0, bhh0, Wih1, Whh1, bih1, bhh1):
  loop = jnp.arange(_N, dtype=jnp.int32)
  npad = _EPAD - _E - _N
  src = jnp.concatenate([edge_index[0], loop,
                         jnp.zeros((npad,), jnp.int32)])
  dst = jnp.concatenate([edge_index[1], loop,
                         jnp.full((npad,), _PAD_NODE, jnp.int32)])

  t_ids = jnp.arange(_SEQ, dtype=jnp.int32)
  h_ids = jnp.arange(_NHID, dtype=jnp.int32)
  c_ids = jnp.arange(_NCOL1, dtype=jnp.int32)
  onehot = (c_ids[None, None, :] ==
            (2 * t_ids[:, None, None] + h_ids[None, :, None])
            ).astype(jnp.float32)
  w1p = jnp.einsum("tkh,thc->tkc", W1, onehot)
  s2 = jnp.zeros((_NCOL2, _NCOL1), jnp.float32)
  s2 = s2.at[:_SEQ].set(jnp.einsum("th,thc->tc", W2[:, :, 0], onehot))
  b1col = jnp.zeros((_NCOL1, 1), jnp.float32)
  b1col = b1col.at[:2 * _SEQ, 0].set(b1.reshape(-1))

  degp = _make_deg()(dst)
  hh = _run_mm(x, w1p)
  hht, dinv = _run_tr(hh, degp)
  norm = _make_norm()(src, dst, dinv)
  msg1 = _make_conv(1)(src, dst, norm, hht)
  ut = _run_mid(msg1, b1col, s2)
  msg2 = _make_conv(2)(src, dst, norm, ut)
  out1, hn = _run_gru(
      msg2, b2, Wih0.T, Whh0.T, bih0.reshape(1, -1), bhh0.reshape(1, -1),
      Wih1.T, Whh1.T, bih1.reshape(1, -1), bhh1.reshape(1, -1))
  return out1.reshape(_SEQ, 1, _H), hn.reshape(2, 1, _H)


# speedup vs baseline: 96.8919x; 1.2309x over previous
import functools

import jax
import jax.numpy as jnp
from jax import lax
from jax.experimental import pallas as pl
from jax.experimental.pallas import tpu as pltpu
from jax.experimental.pallas import tpu_sc as plsc

_N = 10000
_E = 320000
_SEQ = 15
_IN = 128
_NHID = 2
_H = 64

_NCOL1 = 32
_NCOL2 = 16
_NTAB = 10240
_CHUNK = 4000
_NCHUNK = _E // _CHUNK
_NW = 32
_EPG = _E // _NW
_BN = 1000
_NBLK = _N // _BN

_SC_PARAMS = pltpu.CompilerParams(needs_layout_passes=False,
                                  use_tc_tiling_on_sc=False)


@functools.lru_cache(maxsize=None)
def _mesh():
  return plsc.VectorSubcoreMesh(core_axis_name="c", subcore_axis_name="s",
                                num_cores=2, num_subcores=16)


def _wid():
  return lax.axis_index("s") * 2 + lax.axis_index("c")


def _zero_f32(ref, start, count):
  z = jnp.zeros((16,), jnp.float32)

  @pl.loop(0, count)
  def _(i):
    ref[pl.ds(start + i * 16, 16)] = z


@functools.lru_cache(maxsize=None)
def _make_deg():
  @functools.partial(
      pl.kernel,
      out_type=jax.ShapeDtypeStruct((_NW, _N), jnp.float32),
      mesh=_mesh(),
      compiler_params=_SC_PARAMS,
      scratch_types=[
          pltpu.VMEM((_EPG,), jnp.int32),
          pltpu.VMEM((_NTAB,), jnp.float32),
      ],
  )
  def deg_kernel(ei_hbm, degp_hbm, dst_v, acc_v):
    wid = _wid()
    _zero_f32(acc_v, 0, _NTAB // 16)
    pltpu.sync_copy(ei_hbm.at[1, pl.ds(wid * _EPG, _EPG)], dst_v)
    ones = jnp.full((16,), 1.0, jnp.float32)

    @pl.loop(0, _EPG // 16, unroll=8)
    def _(i):
      d = dst_v[pl.ds(i * 16, 16)]
      plsc.addupdate_scatter(acc_v, [d], ones)

    pltpu.sync_copy(acc_v.at[pl.ds(0, _N)], degp_hbm.at[wid])

  return deg_kernel


def _mm_kernel(xb, w1p, hh):
  acc = jnp.zeros((_BN, _NCOL1), jnp.float32)
  for t in range(_SEQ):
    acc = acc + jnp.dot(xb[t], w1p[t], preferred_element_type=jnp.float32)
  hh[...] = acc


def _run_mm(x, w1p):
  return pl.pallas_call(
      _mm_kernel,
      grid=(_NBLK,),
      in_specs=[
          pl.BlockSpec((_SEQ, _BN, _IN), lambda j: (0, j, 0)),
          pl.BlockSpec((_SEQ, _IN, _NCOL1), lambda j: (0, 0, 0)),
      ],
      out_specs=pl.BlockSpec((_BN, _NCOL1), lambda j: (j, 0)),
      out_shape=jax.ShapeDtypeStruct((_N, _NCOL1), jnp.float32),
  )(x, w1p)


def _tr_kernel(hh, degp, hht, dinv):
  deg = jnp.sum(degp[...], axis=0, keepdims=True) + 1.0
  div = lax.rsqrt(deg)
  dinv[...] = div
  hht[...] = hh[...].T * div


def _run_tr(hh, degp):
  return pl.pallas_call(
      _tr_kernel,
      out_shape=[
          jax.ShapeDtypeStruct((_NCOL1, _N), jnp.float32),
          jax.ShapeDtypeStruct((1, _N), jnp.float32),
      ],
  )(hh, degp)


@functools.lru_cache(maxsize=None)
def _make_conv(nsplit):
  chunks_per = _NCHUNK // nsplit

  @functools.partial(
      pl.kernel,
      out_type=jax.ShapeDtypeStruct((_NW, _N), jnp.float32),
      mesh=_mesh(),
      compiler_params=_SC_PARAMS,
      scratch_types=[
          pltpu.VMEM((_NTAB,), jnp.float32),
          pltpu.VMEM((_NTAB,), jnp.float32),
          pltpu.VMEM((2, _CHUNK), jnp.int32),
          pltpu.VMEM((2, _CHUNK), jnp.int32),
          pltpu.SemaphoreType.DMA,
          pltpu.SemaphoreType.DMA,
      ],
  )
  def conv(ei_hbm, tab_hbm, out_hbm, tab_v, acc_v, src_v, dst_v, sem0, sem1):
    wid = _wid()
    col = wid // nsplit
    part = wid % nsplit
    sems = [sem0, sem1]

    def issue(slot, c):
      base = (part * chunks_per + c) * _CHUNK
      pltpu.async_copy(ei_hbm.at[0, pl.ds(base, _CHUNK)], src_v.at[slot],
                       sems[slot])
      pltpu.async_copy(ei_hbm.at[1, pl.ds(base, _CHUNK)], dst_v.at[slot],
                       sems[slot])

    def drain(slot, c):
      base = (part * chunks_per + c) * _CHUNK
      pltpu.make_async_copy(ei_hbm.at[0, pl.ds(base, _CHUNK)],
                            src_v.at[slot], sems[slot]).wait()
      pltpu.make_async_copy(ei_hbm.at[1, pl.ds(base, _CHUNK)],
                            dst_v.at[slot], sems[slot]).wait()

    pltpu.sync_copy(tab_hbm.at[col], tab_v.at[pl.ds(0, _N)])
    if nsplit == 1:
      pltpu.sync_copy(tab_hbm.at[col], acc_v.at[pl.ds(0, _N)])
    else:
      @pl.when(part == 0)
      def _():
        pltpu.sync_copy(tab_hbm.at[col], acc_v.at[pl.ds(0, _N)])

      @pl.when(part != 0)
      def _():
        _zero_f32(acc_v, 0, _N // 16)

    issue(0, 0)

    @pl.loop(0, chunks_per // 2)
    def _(g):
      for b in range(2):
        cidx = g * 2 + b

        @pl.when(cidx + 1 < chunks_per)
        def _():
          issue(1 - b, cidx + 1)

        drain(b, cidx)

        @pl.loop(0, _CHUNK // 16, unroll=8)
        def _(i):
          s = src_v[b, pl.ds(i * 16, 16)]
          d = dst_v[b, pl.ds(i * 16, 16)]
          hv = plsc.load_gather(tab_v, [s])
          plsc.addupdate_scatter(acc_v, [d], hv)

    pltpu.sync_copy(acc_v.at[pl.ds(0, _N)], out_hbm.at[wid])

  return conv


def _mid_kernel(msg1, dinv, b1col, s2, ut):
  h = jnp.maximum(msg1[...] * dinv[...] + b1col[...], 0.0)
  ut[...] = lax.dot_general(
      s2[...], h, (((1,), (0,)), ((), ())),
      preferred_element_type=jnp.float32) * dinv[...]


def _run_mid(msg1, dinv, b1col, s2):
  return pl.pallas_call(
      _mid_kernel,
      out_shape=jax.ShapeDtypeStruct((_NCOL2, _N), jnp.float32),
  )(msg1, dinv, b1col, s2)


def _gru_cell(gi, h, whht, bhh):
  gh = jnp.dot(h, whht[...], preferred_element_type=jnp.float32) + bhh[...]
  r = jax.nn.sigmoid(gi[:, :_H] + gh[:, :_H])
  z = jax.nn.sigmoid(gi[:, _H:2 * _H] + gh[:, _H:2 * _H])
  n = jnp.tanh(gi[:, 2 * _H:] + r * gh[:, 2 * _H:])
  return (1.0 - z) * n + z * h


def _gru_kernel(msg2, dinv, b2, wih0t, whh0t, bih0, bhh0,
                wih1t, whh1t, bih1, bhh1, out1, hn):
  rows = [msg2[2 * t:2 * t + 1, :] + msg2[2 * t + 1:2 * t + 2, :]
          for t in range(_SEQ)]
  seq = jnp.tanh(jnp.concatenate(rows, axis=0) * dinv[...] + b2[...])
  gi0 = jnp.dot(seq, wih0t[...], preferred_element_type=jnp.float32) + bih0[...]
  h = jnp.zeros((1, _H), jnp.float32)
  outs0 = []
  for t in range(_SEQ):
    h = _gru_cell(gi0[t:t + 1, :], h, whh0t, bhh0)
    outs0.append(h)
  out0 = jnp.concatenate(outs0, axis=0)
  h0T = h
  gi1 = jnp.dot(out0, wih1t[...], preferred_element_type=jnp.float32) + bih1[...]
  h = jnp.zeros((1, _H), jnp.float32)
  outs1 = []
  for t in range(_SEQ):
    h = _gru_cell(gi1[t:t + 1, :], h, whh1t, bhh1)
    outs1.append(h)
  out1[...] = jnp.concatenate(outs1, axis=0)
  hn[...] = jnp.concatenate([h0T, h], axis=0)


def _run_gru(msg2, dinv, b2, wih0t, whh0t, bih0, bhh0,
             wih1t, whh1t, bih1, bhh1):
  return pl.pallas_call(
      _gru_kernel,
      out_shape=[
          jax.ShapeDtypeStruct((_SEQ, _H), jnp.float32),
          jax.ShapeDtypeStruct((2, _H), jnp.float32),
      ],
  )(msg2, dinv, b2, wih0t, whh0t, bih0, bhh0, wih1t, whh1t, bih1, bhh1)


def kernel(x, edge_index, W1, b1, W2, b2,
           Wih0, Whh0, bih0, bhh0, Wih1, Whh1, bih1, bhh1):
  t_ids = jnp.arange(_SEQ, dtype=jnp.int32)
  h_ids = jnp.arange(_NHID, dtype=jnp.int32)
  c_ids = jnp.arange(_NCOL1, dtype=jnp.int32)
  onehot = (c_ids[None, None, :] ==
            (2 * t_ids[:, None, None] + h_ids[None, :, None])
            ).astype(jnp.float32)
  w1p = jnp.einsum("tkh,thc->tkc", W1, onehot)
  s2 = jnp.zeros((_NCOL2, _NCOL1), jnp.float32)
  s2 = s2.at[:_SEQ].set(jnp.einsum("th,thc->tc", W2[:, :, 0], onehot))
  b1col = jnp.zeros((_NCOL1, 1), jnp.float32)
  b1col = b1col.at[:2 * _SEQ, 0].set(b1.reshape(-1))

  degp = _make_deg()(edge_index)
  hh = _run_mm(x, w1p)
  hht, dinv = _run_tr(hh, degp)
  msg1 = _make_conv(1)(edge_index, hht)
  ut = _run_mid(msg1, dinv, b1col, s2)
  msg2 = _make_conv(2)(edge_index, ut)
  out1, hn = _run_gru(
      msg2, dinv, b2, Wih0.T, Whh0.T, bih0.reshape(1, -1), bhh0.reshape(1, -1),
      Wih1.T, Whh1.T, bih1.reshape(1, -1), bhh1.reshape(1, -1))
  return out1.reshape(_SEQ, 1, _H), hn.reshape(2, 1, _H)


# speedup vs baseline: 108.0006x; 1.1146x over previous
import functools

import jax
import jax.numpy as jnp
from jax import lax
from jax.experimental import pallas as pl
from jax.experimental.pallas import tpu as pltpu
from jax.experimental.pallas import tpu_sc as plsc

_N = 10000
_E = 320000
_SEQ = 15
_IN = 128
_NHID = 2
_H = 64

_NCOL1 = 32
_NCOL2 = 16
_NTAB = 10240
_CHUNK = 4000
_NCHUNK = _E // _CHUNK
_NW = 32
_EPG = _E // _NW
_BN = 1000
_NBLK = _N // _BN

_SC_PARAMS = pltpu.CompilerParams(needs_layout_passes=False,
                                  use_tc_tiling_on_sc=False)


@functools.lru_cache(maxsize=None)
def _mesh():
  return plsc.VectorSubcoreMesh(core_axis_name="c", subcore_axis_name="s",
                                num_cores=2, num_subcores=16)


def _wid():
  return lax.axis_index("s") * 2 + lax.axis_index("c")


def _zero_f32(ref, start, count):
  z = jnp.zeros((16,), jnp.float32)

  @pl.loop(0, count)
  def _(i):
    ref[pl.ds(start + i * 16, 16)] = z


@functools.lru_cache(maxsize=None)
def _make_deg():
  @functools.partial(
      pl.kernel,
      out_type=jax.ShapeDtypeStruct((_NW, _N), jnp.float32),
      mesh=_mesh(),
      compiler_params=_SC_PARAMS,
      scratch_types=[
          pltpu.VMEM((_EPG,), jnp.int32),
          pltpu.VMEM((_NTAB,), jnp.float32),
      ],
  )
  def deg_kernel(ei_hbm, degp_hbm, dst_v, acc_v):
    wid = _wid()
    _zero_f32(acc_v, 0, _NTAB // 16)
    pltpu.sync_copy(ei_hbm.at[1, pl.ds(wid * _EPG, _EPG)], dst_v)
    ones = jnp.full((16,), 1.0, jnp.float32)

    @pl.loop(0, _EPG // 16, unroll=8)
    def _(i):
      d = dst_v[pl.ds(i * 16, 16)]
      plsc.addupdate_scatter(acc_v, [d], ones)

    pltpu.sync_copy(acc_v.at[pl.ds(0, _N)], degp_hbm.at[wid])

  return deg_kernel


def _mm_kernel(xb, w1p, hh):
  acc = jnp.zeros((_BN, _NCOL1), jnp.float32)
  for t in range(_SEQ):
    acc = acc + jnp.dot(xb[t], w1p[t], preferred_element_type=jnp.float32)
  hh[...] = acc


def _run_mm(x, w1p):
  return pl.pallas_call(
      _mm_kernel,
      grid=(_NBLK,),
      in_specs=[
          pl.BlockSpec((_SEQ, _BN, _IN), lambda j: (0, j, 0)),
          pl.BlockSpec((_SEQ, _IN, _NCOL1), lambda j: (0, 0, 0)),
      ],
      out_specs=pl.BlockSpec((_BN, _NCOL1), lambda j: (j, 0)),
      out_shape=jax.ShapeDtypeStruct((_N, _NCOL1), jnp.float32),
  )(x, w1p)


def _tr_kernel(hh, degp, hht, dinv):
  deg = jnp.sum(degp[...], axis=0, keepdims=True) + 1.0
  div = lax.rsqrt(deg)
  dinv[...] = div
  hht[...] = hh[...].T * div


def _run_tr(hh, degp):
  return pl.pallas_call(
      _tr_kernel,
      out_shape=[
          jax.ShapeDtypeStruct((_NCOL1, _N), jnp.float32),
          jax.ShapeDtypeStruct((1, _N), jnp.float32),
      ],
  )(hh, degp)


@functools.lru_cache(maxsize=None)
def _make_conv(nsplit):
  chunks_per = _NCHUNK // nsplit

  @functools.partial(
      pl.kernel,
      out_type=jax.ShapeDtypeStruct((_NW, _N), jnp.float32),
      mesh=_mesh(),
      compiler_params=_SC_PARAMS,
      scratch_types=[
          pltpu.VMEM((_NTAB,), jnp.float32),
          pltpu.VMEM((_NTAB,), jnp.float32),
          pltpu.VMEM((2, _CHUNK), jnp.int32),
          pltpu.VMEM((2, _CHUNK), jnp.int32),
          pltpu.SemaphoreType.DMA,
          pltpu.SemaphoreType.DMA,
      ],
  )
  def conv(ei_hbm, tab_hbm, out_hbm, tab_v, acc_v, src_v, dst_v, sem0, sem1):
    wid = _wid()
    col = wid // nsplit
    part = wid % nsplit
    sems = [sem0, sem1]

    def issue(slot, c):
      base = (part * chunks_per + c) * _CHUNK
      pltpu.async_copy(ei_hbm.at[0, pl.ds(base, _CHUNK)], src_v.at[slot],
                       sems[slot])
      pltpu.async_copy(ei_hbm.at[1, pl.ds(base, _CHUNK)], dst_v.at[slot],
                       sems[slot])

    def drain(slot, c):
      base = (part * chunks_per + c) * _CHUNK
      pltpu.make_async_copy(ei_hbm.at[0, pl.ds(base, _CHUNK)],
                            src_v.at[slot], sems[slot]).wait()
      pltpu.make_async_copy(ei_hbm.at[1, pl.ds(base, _CHUNK)],
                            dst_v.at[slot], sems[slot]).wait()

    pltpu.sync_copy(tab_hbm.at[col], tab_v.at[pl.ds(0, _N)])
    if nsplit == 1:
      pltpu.sync_copy(tab_hbm.at[col], acc_v.at[pl.ds(0, _N)])
    else:
      @pl.when(part == 0)
      def _():
        pltpu.sync_copy(tab_hbm.at[col], acc_v.at[pl.ds(0, _N)])

      @pl.when(part != 0)
      def _():
        _zero_f32(acc_v, 0, _N // 16)

    issue(0, 0)

    @pl.loop(0, chunks_per // 2)
    def _(g):
      for b in range(2):
        cidx = g * 2 + b

        @pl.when(cidx + 1 < chunks_per)
        def _():
          issue(1 - b, cidx + 1)

        drain(b, cidx)

        @pl.loop(0, _CHUNK // 16, unroll=8)
        def _(i):
          s = src_v[b, pl.ds(i * 16, 16)]
          d = dst_v[b, pl.ds(i * 16, 16)]
          hv = plsc.load_gather(tab_v, [s])
          plsc.addupdate_scatter(acc_v, [d], hv)

    pltpu.sync_copy(acc_v.at[pl.ds(0, _N)], out_hbm.at[wid])

  return conv


def _mid_kernel(msg1, dinv, b1col, s2, ut):
  h = jnp.maximum(msg1[...] * dinv[...] + b1col[...], 0.0)
  ut[...] = lax.dot_general(
      s2[...], h, (((1,), (0,)), ((), ())),
      preferred_element_type=jnp.float32) * dinv[...]


def _run_mid(msg1, dinv, b1col, s2):
  return pl.pallas_call(
      _mid_kernel,
      out_shape=jax.ShapeDtypeStruct((_NCOL2, _N), jnp.float32),
  )(msg1, dinv, b1col, s2)


def _gru_cell(gi, h, whht, bhh):
  gh = jnp.dot(h, whht[...], preferred_element_type=jnp.float32) + bhh[...]
  r = jax.nn.sigmoid(gi[:, :_H] + gh[:, :_H])
  z = jax.nn.sigmoid(gi[:, _H:2 * _H] + gh[:, _H:2 * _H])
  n = jnp.tanh(gi[:, 2 * _H:] + r * gh[:, 2 * _H:])
  return (1.0 - z) * n + z * h


def _gru_kernel(msg2, dinv, b2, wih0, whh0t, bih0, bhh0,
                wih1t, whh1t, bih1, bhh1, out1, hn):
  rows = [msg2[2 * t:2 * t + 1, :] + msg2[2 * t + 1:2 * t + 2, :]
          for t in range(_SEQ)]
  seq = jnp.tanh(jnp.concatenate(rows, axis=0) * dinv[...] + b2[...])
  gi0 = lax.dot_general(
      seq, wih0[...], (((1,), (1,)), ((), ())),
      preferred_element_type=jnp.float32) + bih0[...]
  h = jnp.zeros((1, _H), jnp.float32)
  outs0 = []
  for t in range(_SEQ):
    h = _gru_cell(gi0[t:t + 1, :], h, whh0t, bhh0)
    outs0.append(h)
  out0 = jnp.concatenate(outs0, axis=0)
  h0T = h
  gi1 = jnp.dot(out0, wih1t[...], preferred_element_type=jnp.float32) + bih1[...]
  h = jnp.zeros((1, _H), jnp.float32)
  outs1 = []
  for t in range(_SEQ):
    h = _gru_cell(gi1[t:t + 1, :], h, whh1t, bhh1)
    outs1.append(h)
  out1[...] = jnp.concatenate(outs1, axis=0)
  hn[...] = jnp.concatenate([h0T, h], axis=0)


def _run_gru(msg2, dinv, b2, wih0, whh0t, bih0, bhh0,
             wih1t, whh1t, bih1, bhh1):
  return pl.pallas_call(
      _gru_kernel,
      out_shape=[
          jax.ShapeDtypeStruct((_SEQ, _H), jnp.float32),
          jax.ShapeDtypeStruct((2, _H), jnp.float32),
      ],
  )(msg2, dinv, b2, wih0, whh0t, bih0, bhh0, wih1t, whh1t, bih1, bhh1)


def kernel(x, edge_index, W1, b1, W2, b2,
           Wih0, Whh0, bih0, bhh0, Wih1, Whh1, bih1, bhh1):
  t_ids = jnp.arange(_SEQ, dtype=jnp.int32)
  h_ids = jnp.arange(_NHID, dtype=jnp.int32)
  c_ids = jnp.arange(_NCOL1, dtype=jnp.int32)
  onehot = (c_ids[None, None, :] ==
            (2 * t_ids[:, None, None] + h_ids[None, :, None])
            ).astype(jnp.float32)
  w1p = jnp.einsum("tkh,thc->tkc", W1, onehot)
  s2 = jnp.zeros((_NCOL2, _NCOL1), jnp.float32)
  s2 = s2.at[:_SEQ].set(jnp.einsum("th,thc->tc", W2[:, :, 0], onehot))
  b1col = jnp.zeros((_NCOL1, 1), jnp.float32)
  b1col = b1col.at[:2 * _SEQ, 0].set(b1.reshape(-1))

  degp = _make_deg()(edge_index)
  hh = _run_mm(x, w1p)
  hht, dinv = _run_tr(hh, degp)
  msg1 = _make_conv(1)(edge_index, hht)
  ut = _run_mid(msg1, dinv, b1col, s2)
  msg2 = _make_conv(2)(edge_index, ut)
  out1, hn = _run_gru(
      msg2, dinv, b2, Wih0, Whh0.T, bih0.reshape(1, -1), bhh0.reshape(1, -1),
      Wih1.T, Whh1.T, bih1.reshape(1, -1), bhh1.reshape(1, -1))
  return out1.reshape(_SEQ, 1, _H), hn.reshape(2, 1, _H)


# speedup vs baseline: 184.1277x; 1.7049x over previous
import functools

import jax
import jax.numpy as jnp
from jax import lax
from jax.experimental import pallas as pl
from jax.experimental.pallas import tpu as pltpu
from jax.experimental.pallas import tpu_sc as plsc

_N = 10000
_E = 320000
_SEQ = 15
_IN = 128
_NHID = 2
_H = 64

_NCOL1 = 32
_NCOL2 = 16
_NTAB = 10240
_CHUNK = 3200
_U = 8
_NCHUNK = _E // _CHUNK
_NW = 32
_EPG = _E // _NW
_BN = 1000
_NBLK = _N // _BN

_SC_PARAMS = pltpu.CompilerParams(needs_layout_passes=False,
                                  use_tc_tiling_on_sc=False)


@functools.lru_cache(maxsize=None)
def _mesh():
  return plsc.VectorSubcoreMesh(core_axis_name="c", subcore_axis_name="s",
                                num_cores=2, num_subcores=16)


def _wid():
  return lax.axis_index("s") * 2 + lax.axis_index("c")


def _zero_f32(ref, start, count):
  z = jnp.zeros((16,), jnp.float32)

  @pl.loop(0, count)
  def _(i):
    ref[pl.ds(start + i * 16, 16)] = z


@functools.lru_cache(maxsize=None)
def _make_deg():
  @functools.partial(
      pl.kernel,
      out_type=jax.ShapeDtypeStruct((_NW, _N), jnp.float32),
      mesh=_mesh(),
      compiler_params=_SC_PARAMS,
      scratch_types=[
          pltpu.VMEM((_EPG,), jnp.int32),
          pltpu.VMEM((_NTAB,), jnp.float32),
      ],
  )
  def deg_kernel(ei_hbm, degp_hbm, dst_v, acc_v):
    wid = _wid()
    _zero_f32(acc_v, 0, _NTAB // 16)
    pltpu.sync_copy(ei_hbm.at[1, pl.ds(wid * _EPG, _EPG)], dst_v)
    ones = jnp.full((16,), 1.0, jnp.float32)

    @pl.loop(0, _EPG // 16, unroll=8)
    def _(i):
      d = dst_v[pl.ds(i * 16, 16)]
      plsc.addupdate_scatter(acc_v, [d], ones)

    pltpu.sync_copy(acc_v.at[pl.ds(0, _N)], degp_hbm.at[wid])

  return deg_kernel


def _mm_kernel(xb, w1p, hh):
  acc = jnp.zeros((_BN, _NCOL1), jnp.float32)
  for t in range(_SEQ):
    acc = acc + jnp.dot(xb[t], w1p[t], preferred_element_type=jnp.float32)
  hh[...] = acc


def _run_mm(x, w1p):
  return pl.pallas_call(
      _mm_kernel,
      grid=(_NBLK,),
      in_specs=[
          pl.BlockSpec((_SEQ, _BN, _IN), lambda j: (0, j, 0)),
          pl.BlockSpec((_SEQ, _IN, _NCOL1), lambda j: (0, 0, 0)),
      ],
      out_specs=pl.BlockSpec((_BN, _NCOL1), lambda j: (j, 0)),
      out_shape=jax.ShapeDtypeStruct((_N, _NCOL1), jnp.float32),
  )(x, w1p)


def _tr_kernel(hh, degp, hht, dinv):
  deg = jnp.sum(degp[...], axis=0, keepdims=True) + 1.0
  div = lax.rsqrt(deg)
  dinv[...] = div
  hht[...] = hh[...].T * div


def _run_tr(hh, degp):
  return pl.pallas_call(
      _tr_kernel,
      out_shape=[
          jax.ShapeDtypeStruct((_NCOL1, _N), jnp.float32),
          jax.ShapeDtypeStruct((1, _N), jnp.float32),
      ],
  )(hh, degp)


@functools.lru_cache(maxsize=None)
def _make_conv(nsplit):
  chunks_per = _NCHUNK // nsplit

  @functools.partial(
      pl.kernel,
      out_type=jax.ShapeDtypeStruct((_NW, _N), jnp.float32),
      mesh=_mesh(),
      compiler_params=_SC_PARAMS,
      scratch_types=[
          pltpu.VMEM((_NTAB,), jnp.float32),
          pltpu.VMEM((_NTAB,), jnp.float32),
          pltpu.VMEM((2, _CHUNK), jnp.int32),
          pltpu.VMEM((2, _CHUNK), jnp.int32),
          pltpu.SemaphoreType.DMA,
          pltpu.SemaphoreType.DMA,
      ],
  )
  def conv(ei_hbm, tab_hbm, out_hbm, tab_v, acc_v, src_v, dst_v, sem0, sem1):
    wid = _wid()
    col = wid // nsplit
    part = wid % nsplit
    sems = [sem0, sem1]

    def issue(slot, c):
      base = (part * chunks_per + c) * _CHUNK
      pltpu.async_copy(ei_hbm.at[0, pl.ds(base, _CHUNK)], src_v.at[slot],
                       sems[slot])
      pltpu.async_copy(ei_hbm.at[1, pl.ds(base, _CHUNK)], dst_v.at[slot],
                       sems[slot])

    def drain(slot, c):
      base = (part * chunks_per + c) * _CHUNK
      pltpu.make_async_copy(ei_hbm.at[0, pl.ds(base, _CHUNK)],
                            src_v.at[slot], sems[slot]).wait()
      pltpu.make_async_copy(ei_hbm.at[1, pl.ds(base, _CHUNK)],
                            dst_v.at[slot], sems[slot]).wait()

    pltpu.sync_copy(tab_hbm.at[col], tab_v.at[pl.ds(0, _N)])
    if nsplit == 1:
      pltpu.sync_copy(tab_hbm.at[col], acc_v.at[pl.ds(0, _N)])
    else:
      @pl.when(part == 0)
      def _():
        pltpu.sync_copy(tab_hbm.at[col], acc_v.at[pl.ds(0, _N)])

      @pl.when(part != 0)
      def _():
        _zero_f32(acc_v, 0, _N // 16)

    issue(0, 0)

    @pl.loop(0, chunks_per // 2)
    def _(g):
      for b in range(2):
        cidx = g * 2 + b

        @pl.when(cidx + 1 < chunks_per)
        def _():
          issue(1 - b, cidx + 1)

        drain(b, cidx)

        @pl.loop(0, _CHUNK // (16 * _U))
        def _(i):
          base = i * (16 * _U)
          ss = [src_v[b, pl.ds(base + k * 16, 16)] for k in range(_U)]
          dd = [dst_v[b, pl.ds(base + k * 16, 16)] for k in range(_U)]
          hh = [plsc.load_gather(tab_v, [ss[k]]) for k in range(_U)]
          for k in range(_U):
            plsc.addupdate_scatter(acc_v, [dd[k]], hh[k])

    pltpu.sync_copy(acc_v.at[pl.ds(0, _N)], out_hbm.at[wid])

  return conv


def _mid_kernel(msg1, dinv, b1col, s2, ut):
  h = jnp.maximum(msg1[...] * dinv[...] + b1col[...], 0.0)
  ut[...] = lax.dot_general(
      s2[...], h, (((1,), (0,)), ((), ())),
      preferred_element_type=jnp.float32) * dinv[...]


def _run_mid(msg1, dinv, b1col, s2):
  return pl.pallas_call(
      _mid_kernel,
      out_shape=jax.ShapeDtypeStruct((_NCOL2, _N), jnp.float32),
  )(msg1, dinv, b1col, s2)


def _gru_cell(gi, h, whht, bhh):
  gh = jnp.dot(h, whht[...], preferred_element_type=jnp.float32) + bhh[...]
  r = jax.nn.sigmoid(gi[:, :_H] + gh[:, :_H])
  z = jax.nn.sigmoid(gi[:, _H:2 * _H] + gh[:, _H:2 * _H])
  n = jnp.tanh(gi[:, 2 * _H:] + r * gh[:, 2 * _H:])
  return (1.0 - z) * n + z * h


def _gru_kernel(msg2, dinv, b2, wih0, whh0t, bih0, bhh0,
                wih1t, whh1t, bih1, bhh1, out1, hn):
  rows = [msg2[2 * t:2 * t + 1, :] + msg2[2 * t + 1:2 * t + 2, :]
          for t in range(_SEQ)]
  seq = jnp.tanh(jnp.concatenate(rows, axis=0) * dinv[...] + b2[...])
  gi0 = lax.dot_general(
      seq, wih0[...], (((1,), (1,)), ((), ())),
      preferred_element_type=jnp.float32) + bih0[...]
  h = jnp.zeros((1, _H), jnp.float32)
  outs0 = []
  for t in range(_SEQ):
    h = _gru_cell(gi0[t:t + 1, :], h, whh0t, bhh0)
    outs0.append(h)
  out0 = jnp.concatenate(outs0, axis=0)
  h0T = h
  gi1 = jnp.dot(out0, wih1t[...], preferred_element_type=jnp.float32) + bih1[...]
  h = jnp.zeros((1, _H), jnp.float32)
  outs1 = []
  for t in range(_SEQ):
    h = _gru_cell(gi1[t:t + 1, :], h, whh1t, bhh1)
    outs1.append(h)
  out1[...] = jnp.concatenate(outs1, axis=0)
  hn[...] = jnp.concatenate([h0T, h], axis=0)


def _run_gru(msg2, dinv, b2, wih0, whh0t, bih0, bhh0,
             wih1t, whh1t, bih1, bhh1):
  return pl.pallas_call(
      _gru_kernel,
      out_shape=[
          jax.ShapeDtypeStruct((_SEQ, _H), jnp.float32),
          jax.ShapeDtypeStruct((2, _H), jnp.float32),
      ],
  )(msg2, dinv, b2, wih0, whh0t, bih0, bhh0, wih1t, whh1t, bih1, bhh1)


def kernel(x, edge_index, W1, b1, W2, b2,
           Wih0, Whh0, bih0, bhh0, Wih1, Whh1, bih1, bhh1):
  t_ids = jnp.arange(_SEQ, dtype=jnp.int32)
  h_ids = jnp.arange(_NHID, dtype=jnp.int32)
  c_ids = jnp.arange(_NCOL1, dtype=jnp.int32)
  onehot = (c_ids[None, None, :] ==
            (2 * t_ids[:, None, None] + h_ids[None, :, None])
            ).astype(jnp.float32)
  w1p = jnp.einsum("tkh,thc->tkc", W1, onehot)
  s2 = jnp.zeros((_NCOL2, _NCOL1), jnp.float32)
  s2 = s2.at[:_SEQ].set(jnp.einsum("th,thc->tc", W2[:, :, 0], onehot))
  b1col = jnp.zeros((_NCOL1, 1), jnp.float32)
  b1col = b1col.at[:2 * _SEQ, 0].set(b1.reshape(-1))

  degp = _make_deg()(edge_index)
  hh = _run_mm(x, w1p)
  hht, dinv = _run_tr(hh, degp)
  msg1 = _make_conv(1)(edge_index, hht)
  ut = _run_mid(msg1, dinv, b1col, s2)
  msg2 = _make_conv(2)(edge_index, ut)
  out1, hn = _run_gru(
      msg2, dinv, b2, Wih0, Whh0.T, bih0.reshape(1, -1), bhh0.reshape(1, -1),
      Wih1.T, Whh1.T, bih1.reshape(1, -1), bhh1.reshape(1, -1))
  return out1.reshape(_SEQ, 1, _H), hn.reshape(2, 1, _H)
